# baseline (speedup 1.0000x reference)
import functools

import jax
import jax.numpy as jnp
from jax import lax
from jax.experimental import pallas as pl
from jax.experimental.pallas import tpu as pltpu
from jax.experimental.pallas import tpu_sc as plsc

NUM_OUT = 1024
NUM_WORKERS = 32
CHUNK = 8
HALF_T = 4096
HALF_O = 512
CH_PER_SLAB = 68
WND = 1152
ROW_BLOCK = 96


def _sc_lane_subsample(x_t, n_slabs):
    c, r, t = x_t.shape
    n_tasks = n_slabs * CH_PER_SLAB * 2
    tpw = -(-n_tasks // NUM_WORKERS)
    steps = tpw + (tpw % 2)
    num = 2 * (t - 1)
    den = 2 * (NUM_OUT - 1)
    hlf = NUM_OUT - 1
    mesh = plsc.VectorSubcoreMesh(core_axis_name="c", subcore_axis_name="s")

    @functools.partial(
        pl.kernel,
        mesh=mesh,
        out_type=jax.ShapeDtypeStruct((n_slabs, r, NUM_OUT), x_t.dtype),
        scratch_types=[
            pltpu.VMEM((2, CHUNK, HALF_T), x_t.dtype),
            pltpu.VMEM((2, CHUNK, HALF_O), x_t.dtype),
            pltpu.SemaphoreType.DMA,
            pltpu.SemaphoreType.DMA,
        ],
        compiler_params=pltpu.CompilerParams(needs_layout_passes=False),
    )
    def k(x_hbm, out_hbm, rowbuf, outbuf, isem, osem):
        wid = lax.axis_index("s") * 2 + lax.axis_index("c")
        lane = lax.iota(jnp.int32, 16)

        def task_decode(i):
            tau = jnp.minimum(wid + NUM_WORKERS * i, n_tasks - 1)
            slab = tau // (2 * CH_PER_SLAB)
            rem = tau % (2 * CH_PER_SLAB)
            pos = rem // 2
            half = rem % 2
            rstart = pl.multiple_of(pos * CHUNK, CHUNK)
            return slab, rstart, half

        def in_copy(i, b):
            ci, ri, hi = task_decode(i)
            return pltpu.make_async_copy(
                x_hbm.at[ci, pl.ds(ri, CHUNK), pl.ds(hi * HALF_T, HALF_T)],
                rowbuf.at[b], isem)

        def out_copy(i, b):
            ci, ri, hi = task_decode(i)
            return pltpu.make_async_copy(
                outbuf.at[b],
                out_hbm.at[ci, pl.ds(ri, CHUNK), pl.ds(hi * HALF_O, HALF_O)],
                osem)

        def gather_chunk(i, b):
            _, _, hi = task_decode(i)
            obase = hi * HALF_O
            pbase = hi * HALF_T

            def body(h, _):
                j = h // (HALF_O // 16)
                g = h % (HALF_O // 16)
                o = lane + g * 16 + obase
                pos = (num * o + hlf) // den - pbase
                jv = jnp.zeros((16,), jnp.int32) + j
                vals = plsc.load_gather(rowbuf.at[b], [jv, pos])
                outbuf[b, j, pl.ds(g * 16, 16)] = vals
                return 0

            lax.fori_loop(0, CHUNK * (HALF_O // 16), body, 0, unroll=4)

        in_copy(0, 0).start()
        in_copy(1, 1).start()

        def step(q, _):
            for b in range(2):
                i = q * 2 + b
                in_copy(i, b).wait()

                @pl.when(q > 0)
                def _():
                    out_copy(i - 2, b).wait()

                gather_chunk(i, b)
                out_copy(i, b).start()

                @pl.when(q < steps // 2 - 1)
                def _():
                    in_copy(i + 2, b).start()
            return 0

        lax.fori_loop(0, steps // 2, step, 0)
        out_copy(steps - 2, 0).wait()
        out_copy(steps - 1, 1).wait()

    return k(x_t)


def _tc_body(num, den, hlf, t):
    def body(x_ref, o_ref):
        for k in range(NUM_OUT // 128):
            start = min(1024 * k, t - WND)
            a = x_ref[0, :, start:start + WND]
            l = lax.broadcasted_iota(jnp.int32, (WND, 128), 0)
            o = lax.broadcasted_iota(jnp.int32, (WND, 128), 1) + 128 * k
            pos = (num * o + hlf) // den
            s = (pos - start == l).astype(jnp.float32)
            o_ref[0, :, 128 * k:128 * (k + 1)] = jax.lax.dot_general(
                a, s, (((1,), (0,)), ((), ())),
                preferred_element_type=jnp.float32,
                precision=jax.lax.Precision.HIGHEST)
    return body


def _tc_lane_subsample(x_t, slab_lo, n_slabs):
    c, r, t = x_t.shape
    num = 2 * (t - 1)
    den = 2 * (NUM_OUT - 1)
    hlf = NUM_OUT - 1
    return pl.pallas_call(
        _tc_body(num, den, hlf, t),
        grid=(n_slabs, -(-r // ROW_BLOCK)),
        in_specs=[pl.BlockSpec((1, ROW_BLOCK, t),
                               lambda i, j: (slab_lo + i, j, 0))],
        out_specs=pl.BlockSpec((1, ROW_BLOCK, NUM_OUT),
                               lambda i, j: (i, j, 0)),
        out_shape=jax.ShapeDtypeStruct((n_slabs, r, NUM_OUT), x_t.dtype),
    )(x_t)


def kernel(landmarks):
    x_t = jnp.transpose(landmarks, (2, 1, 0))
    out_sc = _sc_lane_subsample(x_t, 1)
    out_tc = _tc_lane_subsample(x_t, 1, 2)
    out_t = jnp.concatenate([out_sc, out_tc], axis=0)
    return jnp.transpose(out_t, (2, 1, 0))


# speedup vs baseline: 1.3098x; 1.3098x over previous
import functools

import jax
import jax.numpy as jnp
from jax import lax
from jax.experimental import pallas as pl
from jax.experimental.pallas import tpu as pltpu
from jax.experimental.pallas import tpu_sc as plsc

NUM_OUT = 1024
NUM_WORKERS = 32
CHUNK = 8
HALF_T = 4096
HALF_O = 512
CH_PER_SLAB = 68
N_TASKS = 3 * CH_PER_SLAB * 2
NBUF = 3
STEPS = 15


def _sc_lane_subsample(x_t):
    c, r, t = x_t.shape
    num = 2 * (t - 1)
    den = 2 * (NUM_OUT - 1)
    hlf = NUM_OUT - 1
    mesh = plsc.VectorSubcoreMesh(core_axis_name="c", subcore_axis_name="s")

    @functools.partial(
        pl.kernel,
        mesh=mesh,
        out_type=jax.ShapeDtypeStruct((c, r, NUM_OUT), x_t.dtype),
        scratch_types=[
            pltpu.VMEM((NBUF, CHUNK, HALF_T), x_t.dtype),
            pltpu.VMEM((NBUF, CHUNK, HALF_O), x_t.dtype),
            pltpu.SemaphoreType.DMA,
            pltpu.SemaphoreType.DMA,
        ],
        compiler_params=pltpu.CompilerParams(needs_layout_passes=False),
    )
    def k(x_hbm, out_hbm, rowbuf, outbuf, isem, osem):
        wid = lax.axis_index("s") * 2 + lax.axis_index("c")
        lane = lax.iota(jnp.int32, 16)

        def task_decode(i):
            tau = jnp.minimum(wid + NUM_WORKERS * i, N_TASKS - 1)
            slab = tau // (2 * CH_PER_SLAB)
            rem = tau % (2 * CH_PER_SLAB)
            pos = rem // 2
            half = rem % 2
            rstart = pl.multiple_of(pos * CHUNK, CHUNK)
            return slab, rstart, half

        def in_copy(i, b):
            ci, ri, hi = task_decode(i)
            return pltpu.make_async_copy(
                x_hbm.at[ci, pl.ds(ri, CHUNK), pl.ds(hi * HALF_T, HALF_T)],
                rowbuf.at[b], isem)

        def out_copy(i, b):
            ci, ri, hi = task_decode(i)
            return pltpu.make_async_copy(
                outbuf.at[b],
                out_hbm.at[ci, pl.ds(ri, CHUNK), pl.ds(hi * HALF_O, HALF_O)],
                osem)

        def gather_chunk(i, b):
            _, _, hi = task_decode(i)
            obase = hi * HALF_O
            pbase = hi * HALF_T

            def body(h, _):
                j = h // (HALF_O // 16)
                g = h % (HALF_O // 16)
                o = lane + g * 16 + obase
                pos = (num * o + hlf) // den - pbase
                jv = jnp.zeros((16,), jnp.int32) + j
                vals = plsc.load_gather(rowbuf.at[b], [jv, pos])
                outbuf[b, j, pl.ds(g * 16, 16)] = vals
                return 0

            lax.fori_loop(0, CHUNK * (HALF_O // 16), body, 0, unroll=4)

        for b in range(NBUF):
            in_copy(b, b).start()

        def step(q, _):
            for b in range(NBUF):
                i = q * NBUF + b
                in_copy(i, b).wait()

                @pl.when(q > 0)
                def _():
                    out_copy(i - NBUF, b).wait()

                gather_chunk(i, b)
                out_copy(i, b).start()

                @pl.when(q < STEPS // NBUF - 1)
                def _():
                    in_copy(i + NBUF, b).start()
            return 0

        lax.fori_loop(0, STEPS // NBUF, step, 0)
        for b in range(NBUF):
            out_copy(STEPS - NBUF + b, b).wait()

    return k(x_t)


def kernel(landmarks):
    x_t = jnp.transpose(landmarks, (2, 1, 0))
    out_t = _sc_lane_subsample(x_t)
    return jnp.transpose(out_t, (2, 1, 0))


# speedup vs baseline: 1.4217x; 1.0854x over previous
import functools

import jax
import jax.numpy as jnp
from jax import lax
from jax.experimental import pallas as pl
from jax.experimental.pallas import tpu as pltpu
from jax.experimental.pallas import tpu_sc as plsc

NUM_OUT = 1024
NUM_WORKERS = 32
CHUNK = 8
HALF_T = 4096
HALF_O = 512
CH_PER_SLAB = 68
N_TASKS = 3 * CH_PER_SLAB * 2
STEPS = 14


def _sc_lane_subsample(x_t):
    c, r, t = x_t.shape
    num = 2 * (t - 1)
    den = 2 * (NUM_OUT - 1)
    hlf = NUM_OUT - 1
    mesh = plsc.VectorSubcoreMesh(core_axis_name="c", subcore_axis_name="s")

    @functools.partial(
        pl.kernel,
        mesh=mesh,
        out_type=jax.ShapeDtypeStruct((c, r, NUM_OUT), x_t.dtype),
        scratch_types=[
            pltpu.VMEM((2, CHUNK, HALF_T), x_t.dtype),
            pltpu.VMEM((2, CHUNK, HALF_O), x_t.dtype),
            pltpu.SemaphoreType.DMA,
            pltpu.SemaphoreType.DMA,
        ],
        compiler_params=pltpu.CompilerParams(needs_layout_passes=False),
    )
    def k(x_hbm, out_hbm, rowbuf, outbuf, isem, osem):
        wid = lax.axis_index("s") * 2 + lax.axis_index("c")
        lane = lax.iota(jnp.int32, 16)

        def task_decode(i):
            tau = jnp.minimum(wid + NUM_WORKERS * i, N_TASKS - 1)
            slab = tau // (2 * CH_PER_SLAB)
            rem = tau % (2 * CH_PER_SLAB)
            pos = rem // 2
            half = rem % 2
            rstart = pl.multiple_of(pos * CHUNK, CHUNK)
            return slab, rstart, half

        def in_copy(i, b):
            ci, ri, hi = task_decode(i)
            return pltpu.make_async_copy(
                x_hbm.at[ci, pl.ds(ri, CHUNK), pl.ds(hi * HALF_T, HALF_T)],
                rowbuf.at[b], isem)

        def out_copy(i, b):
            ci, ri, hi = task_decode(i)
            return pltpu.make_async_copy(
                outbuf.at[b],
                out_hbm.at[ci, pl.ds(ri, CHUNK), pl.ds(hi * HALF_O, HALF_O)],
                osem)

        def gather_chunk(i, b):
            _, _, hi = task_decode(i)
            obase = hi * HALF_O
            pbase = hi * HALF_T

            def body(h, _):
                j = h // (HALF_O // 16)
                g = h % (HALF_O // 16)
                o = lane + g * 16 + obase
                pos = (num * o + hlf) // den - pbase
                jv = jnp.zeros((16,), jnp.int32) + j
                vals = plsc.load_gather(rowbuf.at[b], [jv, pos])
                outbuf[b, j, pl.ds(g * 16, 16)] = vals
                return 0

            lax.fori_loop(0, CHUNK * (HALF_O // 16), body, 0, unroll=4)

        in_copy(0, 0).start()
        in_copy(1, 1).start()

        def step(q, _):
            for b in range(2):
                i = q * 2 + b
                in_copy(i, b).wait()

                @pl.when(q > 0)
                def _():
                    out_copy(i - 2, b).wait()

                gather_chunk(i, b)
                out_copy(i, b).start()

                @pl.when(q < STEPS // 2 - 1)
                def _():
                    in_copy(i + 2, b).start()
            return 0

        lax.fori_loop(0, STEPS // 2, step, 0)
        out_copy(STEPS - 2, 0).wait()
        out_copy(STEPS - 1, 1).wait()

    return k(x_t)


def kernel(landmarks):
    x_t = jnp.transpose(landmarks, (2, 1, 0))
    out_t = _sc_lane_subsample(x_t)
    return jnp.transpose(out_t, (2, 1, 0))
